# baseline (speedup 1.0000x reference)
import functools

import jax
import jax.numpy as jnp
from jax import lax
from jax.experimental import pallas as pl
from jax.experimental.pallas import tpu as pltpu
from jax.experimental.pallas import tpu_sc as plsc

N = 10000
NC, NS = 2, 16
NW = NC * NS
CHUNK = 128
N_PAD = 10240
RPT = N_PAD // NS
RC = RPT // CHUNK


def _spmm_sc(D, n_chunks):
  mesh = plsc.VectorSubcoreMesh(core_axis_name="c", subcore_axis_name="s")

  @functools.partial(
      pl.kernel,
      out_type=jax.ShapeDtypeStruct((NC, N_PAD, D), jnp.float32),
      mesh=mesh,
      scratch_types=[
          pltpu.VMEM((2, n_chunks // 2, CHUNK), jnp.int32),
          pltpu.VMEM((CHUNK, D), jnp.float32),
          pltpu.VMEM((CHUNK, D), jnp.float32),
          pltpu.VMEM_SHARED((N_PAD, D), jnp.float32),
          pltpu.SemaphoreType.DMA,
          pltpu.SemaphoreType.DMA,
          pltpu.SemaphoreType.DMA,
      ],
  )
  def spmm(table, edges, zeros, out, idx, buf_a, buf_b, acc, gsem, ss_a,
           ss_b):
    c = lax.axis_index("c")
    s = lax.axis_index("s")
    wid = s * NC + c
    rbase = s * RPT

    def gather(j, buf):
      pltpu.async_copy(table.at[idx.at[0, j]], buf, gsem).wait()

    def sstart(j, buf, sem):
      pltpu.async_copy(buf, acc.at[idx.at[1, j]], sem, add=True)

    def swait(buf, sem):
      pltpu.make_async_copy(buf, acc.at[idx.at[1, 0]], sem).wait()

    pltpu.sync_copy(zeros, buf_a)
    for j in range(RC):
      pltpu.sync_copy(buf_a, acc.at[pl.ds(rbase + j * CHUNK, CHUNK)])
    plsc.subcore_barrier()

    for h in range(2):
      pltpu.sync_copy(edges.at[wid, :, pl.ds(h * (n_chunks // 2),
                                             n_chunks // 2)], idx)
      gather(0, buf_a)
      sstart(0, buf_a, ss_a)
      gather(1, buf_b)
      sstart(1, buf_b, ss_b)

      def body(i, carry):
        j0 = 2 * i
        swait(buf_a, ss_a)
        gather(j0, buf_a)
        sstart(j0, buf_a, ss_a)
        swait(buf_b, ss_b)
        gather(j0 + 1, buf_b)
        sstart(j0 + 1, buf_b, ss_b)
        return carry

      lax.fori_loop(1, n_chunks // 4, body, 0)
      swait(buf_a, ss_a)
      swait(buf_b, ss_b)
    plsc.subcore_barrier()
    for j in range(RC):
      pltpu.sync_copy(acc.at[pl.ds(rbase + j * CHUNK, CHUNK)], buf_a)
      pltpu.sync_copy(buf_a, out.at[c, pl.ds(rbase + j * CHUNK, CHUNK)])

  return spmm


def _mm_body(x_ref, w_ref, o_ref):
  o_ref[...] = jnp.dot(x_ref[...], w_ref[...],
                       preferred_element_type=jnp.float32)


def _matmul(x, w):
  m, k = x.shape
  d = w.shape[1]
  bm = m // 4
  return pl.pallas_call(
      _mm_body,
      grid=(4,),
      in_specs=[pl.BlockSpec((bm, k), lambda i: (i, 0)),
                pl.BlockSpec((k, d), lambda i: (0, 0))],
      out_specs=pl.BlockSpec((bm, d), lambda i: (i, 0)),
      out_shape=jax.ShapeDtypeStruct((m, d), jnp.float32),
  )(x, w)


def _fuse_body(relu, pa_ref, pb_ref, b_ref, w_ref, o_ref):
  h = pa_ref[...] + pb_ref[...] + b_ref[...]
  if relu:
    h = jnp.maximum(h, 0.0)
  o_ref[...] = jnp.dot(h, w_ref[...], preferred_element_type=jnp.float32)


def _fuse_matmul(pa, pb, b, w, relu):
  m, k = pa.shape
  d = w.shape[1]
  bm = m // 4
  return pl.pallas_call(
      functools.partial(_fuse_body, relu),
      grid=(4,),
      in_specs=[pl.BlockSpec((bm, k), lambda i: (i, 0)),
                pl.BlockSpec((bm, k), lambda i: (i, 0)),
                pl.BlockSpec((1, k), lambda i: (0, 0)),
                pl.BlockSpec((k, d), lambda i: (0, 0))],
      out_specs=pl.BlockSpec((bm, d), lambda i: (i, 0)),
      out_shape=jax.ShapeDtypeStruct((m, d), jnp.float32),
  )(pa, pb, b.reshape(1, k), w)


def _final_body(pa_ref, pb_ref, b_ref, o_ref):
  h = pa_ref[...] + pb_ref[...] + b_ref[...]
  m = jnp.max(h, axis=1, keepdims=True)
  lse = jnp.log(jnp.sum(jnp.exp(h - m), axis=1, keepdims=True)) + m
  o_ref[...] = h - lse


def _final(pa, pb, b):
  m, d = pa.shape
  bm = m // 4
  return pl.pallas_call(
      _final_body,
      grid=(4,),
      in_specs=[pl.BlockSpec((bm, d), lambda i: (i, 0)),
                pl.BlockSpec((bm, d), lambda i: (i, 0)),
                pl.BlockSpec((1, d), lambda i: (0, 0))],
      out_specs=pl.BlockSpec((bm, d), lambda i: (i, 0)),
      out_shape=jax.ShapeDtypeStruct((m, d), jnp.float32),
  )(pa, pb, b.reshape(1, d))


def kernel(x, adj, W1, b1, W2, b2, W3, b3):
  e = adj.shape[1]
  e_pad = -(-e // (4 * NW * CHUNK)) * (4 * NW * CHUNK)
  n_chunks = e_pad // (NW * CHUNK)
  adj = adj.astype(jnp.int32)
  src = jnp.concatenate([adj[0], jnp.zeros((e_pad - e,), jnp.int32)])
  dst = jnp.concatenate([adj[1], jnp.full((e_pad - e,), N, jnp.int32)])
  edges = jnp.stack([src.reshape(NW, n_chunks, CHUNK),
                     dst.reshape(NW, n_chunks, CHUNK)], axis=1)
  x_pad = jnp.pad(x, ((0, N_PAD - N), (0, 0)))
  z128 = jnp.zeros((CHUNK, 128), jnp.float32)
  spmm128 = _spmm_sc(128, n_chunks)
  W3p = jnp.pad(W3, ((0, 0), (0, 128 - W3.shape[1])))

  t1 = _matmul(x_pad, W1)
  p1 = spmm128(t1, edges, z128)
  t2 = _fuse_matmul(p1[0], p1[1], b1, W2, relu=True)
  p2 = spmm128(t2, edges, z128)
  t3 = _fuse_matmul(p2[0], p2[1], b2, W3p, relu=False)
  p3 = spmm128(t3, edges, z128)
  out = _final(p3[0, :, :64], p3[1, :, :64], b3)
  return out[:N]


# speedup vs baseline: 1.3610x; 1.3610x over previous
import functools

import jax
import jax.numpy as jnp
from jax import lax
from jax.experimental import pallas as pl
from jax.experimental.pallas import tpu as pltpu
from jax.experimental.pallas import tpu_sc as plsc

N = 10000
NC, NS = 2, 16
NW = NC * NS
CHUNK = 128
N_PAD = 10240
RPT = N_PAD // NS
RC = RPT // CHUNK


def _spmm_sc(D, n_chunks):
  mesh = plsc.VectorSubcoreMesh(core_axis_name="c", subcore_axis_name="s")

  @functools.partial(
      pl.kernel,
      out_type=jax.ShapeDtypeStruct((NC, N_PAD, D), jnp.float32),
      mesh=mesh,
      scratch_types=[
          pltpu.VMEM((2, n_chunks, CHUNK), jnp.int32),
          pltpu.VMEM((CHUNK, D), jnp.float32),
          pltpu.VMEM_SHARED((N_PAD, D), jnp.float32),
          pltpu.SemaphoreType.DMA,
      ],
  )
  def spmm(table, edges, zeros, out, idx, buf, acc, sem):
    c = lax.axis_index("c")
    s = lax.axis_index("s")
    wid = s * NC + c
    rbase = s * RPT

    pltpu.sync_copy(edges.at[wid], idx)
    pltpu.sync_copy(zeros, buf)
    for j in range(RC):
      pltpu.sync_copy(buf, acc.at[pl.ds(rbase + j * CHUNK, CHUNK)])
    plsc.subcore_barrier()

    def body(i, carry):
      pltpu.async_copy(table.at[idx.at[0, i]], buf, sem).wait()
      pltpu.sync_copy(buf, acc.at[idx.at[1, i]], add=True)
      return carry

    lax.fori_loop(0, n_chunks, body, 0)
    plsc.subcore_barrier()
    for j in range(RC):
      pltpu.sync_copy(acc.at[pl.ds(rbase + j * CHUNK, CHUNK)], buf)
      pltpu.sync_copy(buf, out.at[c, pl.ds(rbase + j * CHUNK, CHUNK)])

  return spmm


def _mm_body(x_ref, w_ref, o_ref):
  o_ref[...] = jnp.dot(x_ref[...], w_ref[...],
                       preferred_element_type=jnp.float32)


def _matmul(x, w):
  m, k = x.shape
  d = w.shape[1]
  bm = m // 4
  return pl.pallas_call(
      _mm_body,
      grid=(4,),
      in_specs=[pl.BlockSpec((bm, k), lambda i: (i, 0)),
                pl.BlockSpec((k, d), lambda i: (0, 0))],
      out_specs=pl.BlockSpec((bm, d), lambda i: (i, 0)),
      out_shape=jax.ShapeDtypeStruct((m, d), jnp.float32),
  )(x, w)


def _fuse_body(relu, pa_ref, pb_ref, b_ref, w_ref, o_ref):
  h = pa_ref[...] + pb_ref[...] + b_ref[...]
  if relu:
    h = jnp.maximum(h, 0.0)
  o_ref[...] = jnp.dot(h, w_ref[...], preferred_element_type=jnp.float32)


def _fuse_matmul(pa, pb, b, w, relu):
  m, k = pa.shape
  d = w.shape[1]
  bm = m // 4
  return pl.pallas_call(
      functools.partial(_fuse_body, relu),
      grid=(4,),
      in_specs=[pl.BlockSpec((bm, k), lambda i: (i, 0)),
                pl.BlockSpec((bm, k), lambda i: (i, 0)),
                pl.BlockSpec((1, k), lambda i: (0, 0)),
                pl.BlockSpec((k, d), lambda i: (0, 0))],
      out_specs=pl.BlockSpec((bm, d), lambda i: (i, 0)),
      out_shape=jax.ShapeDtypeStruct((m, d), jnp.float32),
  )(pa, pb, b.reshape(1, k), w)


def _final_body(pa_ref, pb_ref, b_ref, o_ref):
  h = pa_ref[...] + pb_ref[...] + b_ref[...]
  m = jnp.max(h, axis=1, keepdims=True)
  lse = jnp.log(jnp.sum(jnp.exp(h - m), axis=1, keepdims=True)) + m
  o_ref[...] = h - lse


def _final(pa, pb, b):
  m, d = pa.shape
  bm = m // 4
  return pl.pallas_call(
      _final_body,
      grid=(4,),
      in_specs=[pl.BlockSpec((bm, d), lambda i: (i, 0)),
                pl.BlockSpec((bm, d), lambda i: (i, 0)),
                pl.BlockSpec((1, d), lambda i: (0, 0))],
      out_specs=pl.BlockSpec((bm, d), lambda i: (i, 0)),
      out_shape=jax.ShapeDtypeStruct((m, d), jnp.float32),
  )(pa, pb, b.reshape(1, d))


def kernel(x, adj, W1, b1, W2, b2, W3, b3):
  e = adj.shape[1]
  e_pad = -(-e // (NW * CHUNK)) * (NW * CHUNK)
  n_chunks = e_pad // (NW * CHUNK)
  adj = adj.astype(jnp.int32)
  src = jnp.concatenate([adj[0], jnp.zeros((e_pad - e,), jnp.int32)])
  dst = jnp.concatenate([adj[1], jnp.full((e_pad - e,), N, jnp.int32)])
  edges = jnp.stack([src.reshape(NW, n_chunks, CHUNK),
                     dst.reshape(NW, n_chunks, CHUNK)], axis=1)
  x_pad = jnp.pad(x, ((0, N_PAD - N), (0, 0)))
  z128 = jnp.zeros((CHUNK, 128), jnp.float32)
  spmm128 = _spmm_sc(128, n_chunks)
  W3p = jnp.pad(W3, ((0, 0), (0, 128 - W3.shape[1])))

  t1 = _matmul(x_pad, W1)
  p1 = spmm128(t1, edges, z128)
  t2 = _fuse_matmul(p1[0], p1[1], b1, W2, relu=True)
  p2 = spmm128(t2, edges, z128)
  t3 = _fuse_matmul(p2[0], p2[1], b2, W3p, relu=False)
  p3 = spmm128(t3, edges, z128)
  out = _final(p3[0, :, :64], p3[1, :, :64], b3)
  return out[:N]


# speedup vs baseline: 1.3621x; 1.0008x over previous
import functools

import jax
import jax.numpy as jnp
from jax import lax
from jax.experimental import pallas as pl
from jax.experimental.pallas import tpu as pltpu
from jax.experimental.pallas import tpu_sc as plsc

N = 10000
NC, NS = 2, 16
NW = NC * NS
CHUNK = 128
N_PAD = 10240
RPT = N_PAD // NS
RC = RPT // CHUNK


def _spmm_sc(D, n_chunks):
  mesh = plsc.VectorSubcoreMesh(core_axis_name="c", subcore_axis_name="s")

  @functools.partial(
      pl.kernel,
      out_type=jax.ShapeDtypeStruct((NC, N_PAD, D), jnp.float32),
      mesh=mesh,
      scratch_types=[
          pltpu.VMEM((2, n_chunks, CHUNK), jnp.int32),
          pltpu.VMEM((CHUNK, D), jnp.float32),
          pltpu.VMEM_SHARED((N_PAD, D), jnp.float32),
          pltpu.SemaphoreType.DMA,
      ],
  )
  def spmm(table, edges, zeros, out, idx, buf, acc, sem):
    c = lax.axis_index("c")
    s = lax.axis_index("s")
    wid = s * NC + c
    rbase = s * RPT

    pltpu.sync_copy(edges.at[wid], idx)
    pltpu.sync_copy(zeros, buf)
    for j in range(RC):
      pltpu.sync_copy(buf, acc.at[pl.ds(rbase + j * CHUNK, CHUNK)])
    plsc.subcore_barrier()

    def body(i, carry):
      pltpu.async_copy(table.at[idx.at[0, i]], buf, sem).wait()
      pltpu.sync_copy(buf, acc.at[idx.at[1, i]], add=True)
      return carry

    lax.fori_loop(0, n_chunks, body, 0)
    plsc.subcore_barrier()
    for j in range(RC):
      pltpu.sync_copy(acc.at[pl.ds(rbase + j * CHUNK, CHUNK)], buf)
      pltpu.sync_copy(buf, out.at[c, pl.ds(rbase + j * CHUNK, CHUNK)])

  return spmm


def _mm_body(x_ref, w_ref, o_ref):
  o_ref[...] = jnp.dot(x_ref[...], w_ref[...],
                       preferred_element_type=jnp.float32)


def _matmul(x, w):
  m, k = x.shape
  d = w.shape[1]
  bm = m // 4
  return pl.pallas_call(
      _mm_body,
      grid=(4,),
      in_specs=[pl.BlockSpec((bm, k), lambda i: (i, 0)),
                pl.BlockSpec((k, d), lambda i: (0, 0))],
      out_specs=pl.BlockSpec((bm, d), lambda i: (i, 0)),
      out_shape=jax.ShapeDtypeStruct((m, d), jnp.float32),
  )(x, w)


def _fuse_body(relu, pa_ref, pb_ref, b_ref, w_ref, o_ref):
  h = pa_ref[...] + pb_ref[...] + b_ref[...]
  if relu:
    h = jnp.maximum(h, 0.0)
  o_ref[...] = jnp.dot(h, w_ref[...], preferred_element_type=jnp.float32)


def _fuse_matmul(pa, pb, b, w, relu):
  m, k = pa.shape
  d = w.shape[1]
  bm = m // 4
  return pl.pallas_call(
      functools.partial(_fuse_body, relu),
      grid=(4,),
      in_specs=[pl.BlockSpec((bm, k), lambda i: (i, 0)),
                pl.BlockSpec((bm, k), lambda i: (i, 0)),
                pl.BlockSpec((1, k), lambda i: (0, 0)),
                pl.BlockSpec((k, d), lambda i: (0, 0))],
      out_specs=pl.BlockSpec((bm, d), lambda i: (i, 0)),
      out_shape=jax.ShapeDtypeStruct((m, d), jnp.float32),
  )(pa, pb, b.reshape(1, k), w)


def _final_body(pa_ref, pb_ref, b_ref, o_ref):
  h = pa_ref[...] + pb_ref[...] + b_ref[...]
  m = jnp.max(h, axis=1, keepdims=True)
  lse = jnp.log(jnp.sum(jnp.exp(h - m), axis=1, keepdims=True)) + m
  o_ref[...] = h - lse


def _final(pa, pb, b):
  m, d = pa.shape
  bm = m // 4
  return pl.pallas_call(
      _final_body,
      grid=(4,),
      in_specs=[pl.BlockSpec((bm, d), lambda i: (i, 0)),
                pl.BlockSpec((bm, d), lambda i: (i, 0)),
                pl.BlockSpec((1, d), lambda i: (0, 0))],
      out_specs=pl.BlockSpec((bm, d), lambda i: (i, 0)),
      out_shape=jax.ShapeDtypeStruct((m, d), jnp.float32),
  )(pa, pb, b.reshape(1, d))


def kernel(x, adj, W1, b1, W2, b2, W3, b3):
  e = adj.shape[1]
  e_pad = -(-e // (NW * CHUNK)) * (NW * CHUNK)
  n_chunks = e_pad // (NW * CHUNK)
  adj = adj.astype(jnp.int32)
  src = jnp.concatenate([adj[0], jnp.zeros((e_pad - e,), jnp.int32)])
  pad_dst = N + jnp.arange(e_pad - e, dtype=jnp.int32) % (N_PAD - N)
  dst = jnp.concatenate([adj[1], pad_dst])
  edges = jnp.stack([src.reshape(NW, n_chunks, CHUNK),
                     dst.reshape(NW, n_chunks, CHUNK)], axis=1)
  x_pad = jnp.pad(x, ((0, N_PAD - N), (0, 0)))
  z128 = jnp.zeros((CHUNK, 128), jnp.float32)
  spmm128 = _spmm_sc(128, n_chunks)
  W3p = jnp.pad(W3, ((0, 0), (0, 128 - W3.shape[1])))

  t1 = _matmul(x_pad, W1)
  p1 = spmm128(t1, edges, z128)
  t2 = _fuse_matmul(p1[0], p1[1], b1, W2, relu=True)
  p2 = spmm128(t2, edges, z128)
  t3 = _fuse_matmul(p2[0], p2[1], b2, W3p, relu=False)
  p3 = spmm128(t3, edges, z128)
  out = _final(p3[0, :, :64], p3[1, :, :64], b3)
  return out[:N]
